# Initial kernel scaffold; baseline (speedup 1.0000x reference)
import functools

import jax
import jax.numpy as jnp
from jax import lax
from jax.experimental import pallas as pl
from jax.experimental.pallas import tpu as pltpu
from jax.experimental.pallas import tpu_sc as plsc

N = 10000
E = 320000
D_IN = 128
H = 128
C = 40

NP = 10016
C_PAD = 48
NTILES = 32
NCHUNK = 80
E_PAD = NTILES * NCHUNK * 128
ROWS_PER_SUB = NP // 16
HALF_ROWS = ROWS_PER_SUB // 2

_MESH = plsc.VectorSubcoreMesh(core_axis_name="c", subcore_axis_name="s")


@functools.partial(
    pl.kernel,
    out_type=jax.ShapeDtypeStruct((2, 80, 128), jnp.float32),
    mesh=_MESH,
    scratch_types=[
        pltpu.VMEM((NCHUNK * 128,), jnp.int32),
        pltpu.VMEM((80, 128), jnp.float32),
        pltpu.VMEM((1, 80), jnp.int32),
        pltpu.VMEM((5, 128), jnp.float32),
        pltpu.VMEM_SHARED((80, 128), jnp.float32),
    ],
)
def _deg_kernel(dst_hbm, out_hbm, dst_v, deg_v, rowidx_v, tmp_v, sdeg):
    cid = lax.axis_index("c")
    sid = lax.axis_index("s")
    tile = cid * 16 + sid

    pltpu.sync_copy(dst_hbm.at[tile], dst_v)

    zero16 = jnp.zeros((16,), jnp.float32)

    def zero_deg(j, carry):
        deg_v[j // 8, pl.ds((j % 8) * 16, 16)] = zero16
        return carry

    lax.fori_loop(0, 640, zero_deg, 0)

    def zero_tmp(j, carry):
        tmp_v[j // 8, pl.ds((j % 8) * 16, 16)] = zero16
        return carry

    lax.fori_loop(0, 40, zero_tmp, 0)

    def fill_rowidx(j, carry):
        rowidx_v[0, pl.ds(j * 16, 16)] = lax.iota(jnp.int32, 16) + j * 16
        return carry

    lax.fori_loop(0, 5, fill_rowidx, 0)

    ones16 = jnp.ones((16,), jnp.float32)

    def count(k, carry):
        idx = dst_v[pl.ds(k * 16, 16)]
        plsc.addupdate_scatter(deg_v, [idx >> 7, idx & 127], ones16)
        return carry

    lax.fori_loop(0, NCHUNK * 8, count, 0)

    pltpu.sync_copy(tmp_v, sdeg.at[pl.ds(5 * sid, 5)])
    plsc.subcore_barrier()
    pltpu.sync_copy(deg_v, sdeg.at[rowidx_v.at[0]], add=True)
    plsc.subcore_barrier()
    pltpu.sync_copy(sdeg.at[pl.ds(5 * sid, 5)], tmp_v)
    pltpu.sync_copy(tmp_v, out_hbm.at[cid, pl.ds(5 * sid, 5)])


def _make_scatter(D):
    @functools.partial(
        pl.kernel,
        out_type=jax.ShapeDtypeStruct((2, NP, D), jnp.float32),
        mesh=_MESH,
        scratch_types=[
            pltpu.VMEM((NCHUNK, 128), jnp.int32),
            pltpu.VMEM((NCHUNK, 128), jnp.int32),
            pltpu.VMEM((128, D), jnp.float32),
            pltpu.VMEM((128, D), jnp.float32),
            pltpu.VMEM((HALF_ROWS, D), jnp.float32),
            pltpu.VMEM_SHARED((NP, D), jnp.float32),
            pltpu.SemaphoreType.DMA,
            pltpu.SemaphoreType.DMA,
        ],
    )
    def scat(p_hbm, src_hbm, dst_hbm, zeros_hbm, out_hbm,
             src_v, dst_v, buf0, buf1, stage, acc, sem0, sem1):
        cid = lax.axis_index("c")
        sid = lax.axis_index("s")
        tile = cid * 16 + sid
        base = sid * ROWS_PER_SUB

        pltpu.sync_copy(zeros_hbm, stage)
        pltpu.sync_copy(stage, acc.at[pl.ds(base, HALF_ROWS)])
        pltpu.sync_copy(stage, acc.at[pl.ds(base + HALF_ROWS, HALF_ROWS)])

        pltpu.sync_copy(src_hbm.at[tile], src_v)
        pltpu.sync_copy(dst_hbm.at[tile], dst_v)
        plsc.subcore_barrier()

        pltpu.async_copy(p_hbm.at[src_v.at[0]], buf0, sem0)

        def body(i, carry):
            c0 = 2 * i
            pltpu.async_copy(p_hbm.at[src_v.at[c0 + 1]], buf1, sem1)
            pltpu.make_async_copy(p_hbm.at[src_v.at[c0]], buf0, sem0).wait()
            pltpu.sync_copy(buf0, acc.at[dst_v.at[c0]], add=True)

            @pl.when(c0 + 2 < NCHUNK)
            def _():
                pltpu.async_copy(p_hbm.at[src_v.at[c0 + 2]], buf0, sem0)

            pltpu.make_async_copy(p_hbm.at[src_v.at[c0 + 1]], buf1, sem1).wait()
            pltpu.sync_copy(buf1, acc.at[dst_v.at[c0 + 1]], add=True)
            return carry

        lax.fori_loop(0, NCHUNK // 2, body, 0)
        plsc.subcore_barrier()

        pltpu.sync_copy(acc.at[pl.ds(base, HALF_ROWS)], stage)
        pltpu.sync_copy(stage, out_hbm.at[cid, pl.ds(base, HALF_ROWS)])
        pltpu.sync_copy(acc.at[pl.ds(base + HALF_ROWS, HALF_ROWS)], stage)
        pltpu.sync_copy(stage, out_hbm.at[cid, pl.ds(base + HALF_ROWS, HALF_ROWS)])

    return scat


_scatter128 = _make_scatter(128)
_scatter48 = _make_scatter(C_PAD)


_BLK = 1000


def _dis_body(deg_ref, dis_ref):
    deg = deg_ref[0] + deg_ref[1] + 1.0
    dis_ref[...] = lax.rsqrt(deg)


def _dis_from_deg(deg2):
    return pl.pallas_call(
        _dis_body,
        out_shape=jax.ShapeDtypeStruct((80, 128), jnp.float32),
    )(deg2)


def _pre_body(x_ref, w_ref, dis_ref, out_ref):
    h = jnp.dot(x_ref[...], w_ref[...], preferred_element_type=jnp.float32)
    out_ref[...] = h * dis_ref[...]


def _pre(x, W1, dis_col):
    return pl.pallas_call(
        _pre_body,
        grid=(N // _BLK,),
        in_specs=[
            pl.BlockSpec((_BLK, D_IN), lambda i: (i, 0)),
            pl.BlockSpec((D_IN, H), lambda i: (0, 0)),
            pl.BlockSpec((_BLK, 1), lambda i: (i, 0)),
        ],
        out_specs=pl.BlockSpec((_BLK, H), lambda i: (i, 0)),
        out_shape=jax.ShapeDtypeStruct((N, H), jnp.float32),
    )(x, W1, dis_col)


def _mid_a_body(s0_ref, s1_ref, p_ref, dis_ref, b_ref, y_ref, sum_ref, sq_ref):
    i = pl.program_id(0)
    y = (s0_ref[...] + s1_ref[...] + p_ref[...]) * dis_ref[...] + b_ref[...]
    y_ref[...] = y
    part_sum = jnp.sum(y, axis=0, keepdims=True)
    part_sq = jnp.sum(y * y, axis=0, keepdims=True)

    @pl.when(i == 0)
    def _():
        sum_ref[...] = part_sum
        sq_ref[...] = part_sq

    @pl.when(i > 0)
    def _():
        sum_ref[...] += part_sum
        sq_ref[...] += part_sq


def _mid_a(s0, s1, p, dis_col, b):
    return pl.pallas_call(
        _mid_a_body,
        grid=(N // _BLK,),
        in_specs=[
            pl.BlockSpec((_BLK, H), lambda i: (i, 0)),
            pl.BlockSpec((_BLK, H), lambda i: (i, 0)),
            pl.BlockSpec((_BLK, H), lambda i: (i, 0)),
            pl.BlockSpec((_BLK, 1), lambda i: (i, 0)),
            pl.BlockSpec((1, H), lambda i: (0, 0)),
        ],
        out_specs=[
            pl.BlockSpec((_BLK, H), lambda i: (i, 0)),
            pl.BlockSpec((1, H), lambda i: (0, 0)),
            pl.BlockSpec((1, H), lambda i: (0, 0)),
        ],
        out_shape=[
            jax.ShapeDtypeStruct((N, H), jnp.float32),
            jax.ShapeDtypeStruct((1, H), jnp.float32),
            jax.ShapeDtypeStruct((1, H), jnp.float32),
        ],
    )(s0, s1, p, dis_col, b)


def _mid_b_body(y_ref, sum_ref, sq_ref, g_ref, be_ref, w_ref, dis_ref, out_ref):
    mean = sum_ref[...] * (1.0 / N)
    var = sq_ref[...] * (1.0 / N) - mean * mean
    inv = lax.rsqrt(var + 1e-5)
    z = (y_ref[...] - mean) * inv * g_ref[...] + be_ref[...]
    z = jnp.maximum(z, 0.0)
    h = jnp.dot(z, w_ref[...], preferred_element_type=jnp.float32)
    out_ref[...] = h * dis_ref[...]


def _mid_b(y, ysum, ysq, g, be, W, dis_col, d_out):
    return pl.pallas_call(
        _mid_b_body,
        grid=(N // _BLK,),
        in_specs=[
            pl.BlockSpec((_BLK, H), lambda i: (i, 0)),
            pl.BlockSpec((1, H), lambda i: (0, 0)),
            pl.BlockSpec((1, H), lambda i: (0, 0)),
            pl.BlockSpec((1, H), lambda i: (0, 0)),
            pl.BlockSpec((1, H), lambda i: (0, 0)),
            pl.BlockSpec((H, d_out), lambda i: (0, 0)),
            pl.BlockSpec((_BLK, 1), lambda i: (i, 0)),
        ],
        out_specs=pl.BlockSpec((_BLK, d_out), lambda i: (i, 0)),
        out_shape=jax.ShapeDtypeStruct((N, d_out), jnp.float32),
    )(y, ysum, ysq, g, be, W, dis_col)


def _final_body(s0_ref, s1_ref, p_ref, dis_ref, b_ref, out_ref):
    y = (s0_ref[...] + s1_ref[...] + p_ref[...]) * dis_ref[...]
    y = y[:, :C] + b_ref[...]
    m = jnp.max(y, axis=1, keepdims=True)
    e = jnp.exp(y - m)
    lse = jnp.log(jnp.sum(e, axis=1, keepdims=True))
    out_ref[...] = y - m - lse


def _final(s0, s1, p, dis_col, b3):
    return pl.pallas_call(
        _final_body,
        grid=(N // _BLK,),
        in_specs=[
            pl.BlockSpec((_BLK, C_PAD), lambda i: (i, 0)),
            pl.BlockSpec((_BLK, C_PAD), lambda i: (i, 0)),
            pl.BlockSpec((_BLK, C_PAD), lambda i: (i, 0)),
            pl.BlockSpec((_BLK, 1), lambda i: (i, 0)),
            pl.BlockSpec((1, C), lambda i: (0, 0)),
        ],
        out_specs=pl.BlockSpec((_BLK, C), lambda i: (i, 0)),
        out_shape=jax.ShapeDtypeStruct((N, C), jnp.float32),
    )(s0, s1, p, dis_col, b3)


def kernel(x, edge_index, W1, b1, W2, b2, W3, b3, g1, be1, g2, be2):
    src = edge_index[0].astype(jnp.int32)
    dst = edge_index[1].astype(jnp.int32)
    pad = jnp.full((E_PAD - E,), N, dtype=jnp.int32)
    src_rows = jnp.concatenate([src, pad]).reshape(NTILES, NCHUNK, 128)
    dst_rows = jnp.concatenate([dst, pad]).reshape(NTILES, NCHUNK, 128)
    dst_flat = dst_rows.reshape(NTILES, NCHUNK * 128)

    zeros128 = jnp.zeros((HALF_ROWS, 128), jnp.float32)
    zeros48 = jnp.zeros((HALF_ROWS, C_PAD), jnp.float32)

    deg2 = _deg_kernel(dst_flat)
    dis2d = _dis_from_deg(deg2)
    dis_col = dis2d.reshape(-1)[:N].reshape(N, 1)

    b1r = b1.reshape(1, H)
    b2r = b2.reshape(1, H)
    b3r = b3.reshape(1, C)
    g1r, be1r = g1.reshape(1, H), be1.reshape(1, H)
    g2r, be2r = g2.reshape(1, H), be2.reshape(1, H)

    p1 = _pre(x, W1, dis_col)
    p1_pad = jnp.pad(p1, ((0, NP - N), (0, 0)))
    part1 = _scatter128(p1_pad, src_rows, dst_rows, zeros128)
    y1, s1sum, s1sq = _mid_a(part1[0, :N], part1[1, :N], p1, dis_col, b1r)
    p2 = _mid_b(y1, s1sum, s1sq, g1r, be1r, W2, dis_col, H)
    p2_pad = jnp.pad(p2, ((0, NP - N), (0, 0)))
    part2 = _scatter128(p2_pad, src_rows, dst_rows, zeros128)
    y2, s2sum, s2sq = _mid_a(part2[0, :N], part2[1, :N], p2, dis_col, b2r)
    p3 = _mid_b(y2, s2sum, s2sq, g2r, be2r, W3, dis_col, C)
    p3_pad = jnp.pad(p3, ((0, NP - N), (0, C_PAD - C)))
    part3 = _scatter48(p3_pad, src_rows, dst_rows, zeros48)
    return _final(part3[0, :N], part3[1, :N], p3_pad[:N], dis_col, b3r)


# speedup vs baseline: 3.0597x; 3.0597x over previous
import functools

import jax
import jax.numpy as jnp
from jax import lax
from jax.experimental import pallas as pl
from jax.experimental.pallas import tpu as pltpu
from jax.experimental.pallas import tpu_sc as plsc

N = 10000
E = 320000
D_IN = 128
H = 128
C = 40

NP = 10240
E_PAD = 327680
ROWS_PER_SUB = NP // 16

NR_HALF = 5120
NR_ACC = 5248
NCHUNK_NS = E_PAD // (16 * 128)

_MESH = plsc.VectorSubcoreMesh(core_axis_name="c", subcore_axis_name="s")


@functools.partial(
    pl.kernel,
    out_type=jax.ShapeDtypeStruct((2, NR_ACC, 1, H), jnp.float32),
    mesh=_MESH,
    scratch_types=[
        pltpu.VMEM((128,), jnp.int32),
        pltpu.VMEM((128,), jnp.int32),
        pltpu.VMEM((128, 1, H), jnp.float32),
        pltpu.VMEM_SHARED((NR_ACC, 1, H), jnp.float32),
        pltpu.SemaphoreType.DMA,
    ],
)
def _scatter128(p_hbm, src_hbm, dst_hbm, zeros_hbm, out_hbm,
                src_c, dst_c, buf0, acc, sem0):
    cid = lax.axis_index("c")
    sid = lax.axis_index("s")

    @pl.when(sid == 0)
    def _():
        pltpu.sync_copy(zeros_hbm, acc)

    plsc.subcore_barrier()

    def body(c, carry):
        pltpu.sync_copy(src_hbm.at[pl.ds((sid * NCHUNK_NS + c) * 128, 128)], src_c)
        pltpu.sync_copy(
            dst_hbm.at[pl.ds(((cid * 16 + sid) * NCHUNK_NS + c) * 128, 128)],
            dst_c)
        pltpu.async_copy(p_hbm.at[src_c], buf0, sem0).wait()
        pltpu.sync_copy(buf0, acc.at[dst_c], add=True)
        return carry

    lax.fori_loop(0, NCHUNK_NS, body, 0)
    plsc.subcore_barrier()

    @pl.when(sid == 0)
    def _():
        pltpu.sync_copy(acc, out_hbm.at[cid])


_BLK = 1000


def _dis_body(s_ref, out_ref):
    out_ref[...] = lax.rsqrt(s_ref[...] + 1.0)


def _dis(s):
    return pl.pallas_call(
        _dis_body,
        grid=(N // _BLK,),
        in_specs=[pl.BlockSpec((_BLK, H), lambda i: (i, 0))],
        out_specs=pl.BlockSpec((_BLK, H), lambda i: (i, 0)),
        out_shape=jax.ShapeDtypeStruct((N, H), jnp.float32),
    )(s)


def _pre_body(x_ref, w_ref, dis_ref, out_ref):
    h = jnp.dot(x_ref[...], w_ref[...], preferred_element_type=jnp.float32)
    out_ref[...] = h * dis_ref[...]


def _pre(x, W1, dis_col):
    return pl.pallas_call(
        _pre_body,
        grid=(N // _BLK,),
        in_specs=[
            pl.BlockSpec((_BLK, D_IN), lambda i: (i, 0)),
            pl.BlockSpec((D_IN, H), lambda i: (0, 0)),
            pl.BlockSpec((_BLK, 1), lambda i: (i, 0)),
        ],
        out_specs=pl.BlockSpec((_BLK, H), lambda i: (i, 0)),
        out_shape=jax.ShapeDtypeStruct((N, H), jnp.float32),
    )(x, W1, dis_col)


def _mid_a_body(s_ref, p_ref, dis_ref, b_ref, y_ref, sum_ref, sq_ref):
    i = pl.program_id(0)
    y = (s_ref[...] + p_ref[...]) * dis_ref[...] + b_ref[...]
    y_ref[...] = y
    part_sum = jnp.sum(y, axis=0, keepdims=True)
    part_sq = jnp.sum(y * y, axis=0, keepdims=True)

    @pl.when(i == 0)
    def _():
        sum_ref[...] = part_sum
        sq_ref[...] = part_sq

    @pl.when(i > 0)
    def _():
        sum_ref[...] += part_sum
        sq_ref[...] += part_sq


def _mid_a(s, p, dis_col, b):
    return pl.pallas_call(
        _mid_a_body,
        grid=(N // _BLK,),
        in_specs=[
            pl.BlockSpec((_BLK, H), lambda i: (i, 0)),
            pl.BlockSpec((_BLK, H), lambda i: (i, 0)),
            pl.BlockSpec((_BLK, 1), lambda i: (i, 0)),
            pl.BlockSpec((1, H), lambda i: (0, 0)),
        ],
        out_specs=[
            pl.BlockSpec((_BLK, H), lambda i: (i, 0)),
            pl.BlockSpec((1, H), lambda i: (0, 0)),
            pl.BlockSpec((1, H), lambda i: (0, 0)),
        ],
        out_shape=[
            jax.ShapeDtypeStruct((N, H), jnp.float32),
            jax.ShapeDtypeStruct((1, H), jnp.float32),
            jax.ShapeDtypeStruct((1, H), jnp.float32),
        ],
    )(s, p, dis_col, b)


def _mid_b_body(y_ref, sum_ref, sq_ref, g_ref, be_ref, w_ref, dis_ref, out_ref):
    mean = sum_ref[...] * (1.0 / N)
    var = sq_ref[...] * (1.0 / N) - mean * mean
    inv = lax.rsqrt(var + 1e-5)
    z = (y_ref[...] - mean) * inv * g_ref[...] + be_ref[...]
    z = jnp.maximum(z, 0.0)
    h = jnp.dot(z, w_ref[...], preferred_element_type=jnp.float32)
    out_ref[...] = h * dis_ref[...]


def _mid_b(y, ysum, ysq, g, be, W, dis_col, d_out):
    return pl.pallas_call(
        _mid_b_body,
        grid=(N // _BLK,),
        in_specs=[
            pl.BlockSpec((_BLK, H), lambda i: (i, 0)),
            pl.BlockSpec((1, H), lambda i: (0, 0)),
            pl.BlockSpec((1, H), lambda i: (0, 0)),
            pl.BlockSpec((1, H), lambda i: (0, 0)),
            pl.BlockSpec((1, H), lambda i: (0, 0)),
            pl.BlockSpec((H, d_out), lambda i: (0, 0)),
            pl.BlockSpec((_BLK, 1), lambda i: (i, 0)),
        ],
        out_specs=pl.BlockSpec((_BLK, d_out), lambda i: (i, 0)),
        out_shape=jax.ShapeDtypeStruct((N, d_out), jnp.float32),
    )(y, ysum, ysq, g, be, W, dis_col)


def _final_body(s_ref, p_ref, dis_ref, b_ref, out_ref):
    y = (s_ref[...] + p_ref[...]) * dis_ref[...]
    y = y[:, :C] + b_ref[...]
    m = jnp.max(y, axis=1, keepdims=True)
    e = jnp.exp(y - m)
    lse = jnp.log(jnp.sum(e, axis=1, keepdims=True))
    out_ref[...] = y - m - lse


def _final(s, p, dis_col, b3):
    return pl.pallas_call(
        _final_body,
        grid=(N // _BLK,),
        in_specs=[
            pl.BlockSpec((_BLK, H), lambda i: (i, 0)),
            pl.BlockSpec((_BLK, H), lambda i: (i, 0)),
            pl.BlockSpec((_BLK, 1), lambda i: (i, 0)),
            pl.BlockSpec((1, C), lambda i: (0, 0)),
        ],
        out_specs=pl.BlockSpec((_BLK, C), lambda i: (i, 0)),
        out_shape=jax.ShapeDtypeStruct((N, C), jnp.float32),
    )(s, p, dis_col, b3)


def kernel(x, edge_index, W1, b1, W2, b2, W3, b3, g1, be1, g2, be2):
    src = edge_index[0].astype(jnp.int32)
    dst = edge_index[1].astype(jnp.int32)
    pad = jnp.full((E_PAD - E,), N, dtype=jnp.int32)
    src_flat = jnp.concatenate([src, pad])
    dst_flat = jnp.concatenate([dst, pad])

    dst0 = jnp.where(dst_flat < NR_HALF, dst_flat, NR_HALF)
    dst1 = jnp.where(dst_flat >= NR_HALF, dst_flat - NR_HALF, NR_HALF)
    dst1 = jnp.where(dst1 > NR_HALF, NR_HALF, dst1)
    dst_ns = jnp.concatenate([dst0, dst1])

    zeros128 = jnp.zeros((NR_ACC, 1, H), jnp.float32)
    ones_p = jnp.ones((NP, H), jnp.float32)

    Ws = jnp.stack([W2, jnp.pad(W3, ((0, 0), (0, H - C))),
                    jnp.zeros((H, H), jnp.float32)])
    b3p = jnp.pad(b3, (0, H - C)).reshape(1, H)
    bs = jnp.stack([b1.reshape(1, H), b2.reshape(1, H), b3p])
    gs = jnp.stack([g1.reshape(1, H), g2.reshape(1, H),
                    jnp.ones((1, H), jnp.float32)])
    bes = jnp.stack([be1.reshape(1, H), be2.reshape(1, H),
                     jnp.zeros((1, H), jnp.float32)])

    niter = jnp.int32(4) + jnp.min(dst) * 0

    def cond(st):
        return st[0] < niter

    def body(st):
        i, p_pad, _, _, dis_col = st
        part = _scatter128(p_pad.reshape(NP, 1, H), src_flat, dst_ns, zeros128)
        s = jnp.concatenate(
            [part[0, :NR_HALF, 0], part[1, :N - NR_HALF, 0]], axis=0)

        def f_deg(_):
            dcol = _dis(s)[:, :1]
            pn = _pre(x, W1, dcol)
            return jnp.pad(pn, ((0, NP - N), (0, 0))), dcol

        def f_layer(_):
            j = i - 1
            W = lax.dynamic_index_in_dim(Ws, j, keepdims=False)
            b = lax.dynamic_index_in_dim(bs, j, keepdims=False)
            g = lax.dynamic_index_in_dim(gs, j, keepdims=False)
            be = lax.dynamic_index_in_dim(bes, j, keepdims=False)
            y, ssum, ssq = _mid_a(s, p_pad[:N], dis_col, b)
            pn = _mid_b(y, ssum, ssq, g, be, W, dis_col, H)
            return jnp.pad(pn, ((0, NP - N), (0, 0))), dis_col

        pn_pad, dis_new = lax.cond(i == 0, f_deg, f_layer, 0)
        return (i + 1, pn_pad, p_pad, s, dis_new)

    init = (jnp.int32(0), ones_p, ones_p,
            jnp.zeros((N, H), jnp.float32), jnp.zeros((N, 1), jnp.float32))
    _, _, p3_pad, s3, dis_col = lax.while_loop(cond, body, init)

    return _final(s3, p3_pad[:N], dis_col, b3.reshape(1, C))
